# baseline (speedup 1.0000x reference)
import functools

import jax
import jax.numpy as jnp
from jax.experimental import pallas as pl
from jax.experimental.pallas import tpu as pltpu

_B, _P, _C = 32, 8732, 21
_PBLK = 1280
_NJ = (_P + _PBLK - 1) // _PBLK
_GB = 8
_NG = _B // _GB
_EPS = 1e-7


def _body(lam_ref, c_ref, t_ref, ci_ref, lo_ref, ls_ref, li_ref,
          out_ref,
          a_ab, a_lc, a_rc, a_ll, a_rl, a_wi, a_wl, a_wr):
    g = pl.program_id(0)
    j = pl.program_id(1)

    @pl.when((g == 0) & (j == 0))
    def _init():
        for a in (a_ab, a_lc, a_rc, a_ll, a_rl, a_wi, a_wl, a_wr):
            a[...] = jnp.zeros_like(a)

    lam = lam_ref[0, 0]
    lane = jax.lax.broadcasted_iota(jnp.int32, (1, _PBLK), 1)
    valid = (j * _PBLK + lane) < _P


    c0 = c_ref[0]
    t0 = t_ref[0]
    cmax = c0
    tmax = t0
    for k in range(1, _C):
        cmax = jnp.maximum(cmax, c_ref[k])
        tmax = jnp.maximum(tmax, t_ref[k])
    left = cmax > c0
    right = tmax > t0
    bi = left & right & valid
    bl = left & ~right & valid
    br = right & ~left & valid
    wi = bi.astype(jnp.float32)
    wl = bl.astype(jnp.float32)
    wr = br.astype(jnp.float32)

    v_r = jnp.zeros((_GB, _PBLK), jnp.float32)
    for k in range(_C):
        ck = c_ref[k]
        tk = t_ref[k]
        ins = jnp.where(valid, ci_ref[k], 0.5) + _EPS
        mixed = lam * ck + (1.0 - lam) * tk + _EPS
        den = jnp.where(bi, mixed, ins)
        num = jnp.where(bi, ins,
                        jnp.where(bl, ck + _EPS,
                                  jnp.where(br, tk + _EPS, den)))
        coef = jnp.where(bi, ins - mixed,
                         jnp.where(bl, ck + _EPS,
                                   jnp.where(br, tk + _EPS, 0.0)))
        v_r = v_r + coef * jnp.log(num / den)

    lo = lo_ref[...].reshape(4 * _GB, _PBLK)
    ls = ls_ref[...].reshape(4 * _GB, _PBLK)
    li = li_ref[...].reshape(4 * _GB, _PBLK)
    dl = jnp.where(valid, li - lo, 0.0)
    dr = jnp.where(valid, li - ls, 0.0)
    ri = jax.lax.broadcasted_iota(jnp.int32, (_GB, 4 * _GB), 0)
    cj = jax.lax.broadcasted_iota(jnp.int32, (_GB, 4 * _GB), 1)
    red = ((cj // 4) == ri).astype(jnp.float32)
    dn = (((1,), (0,)), ((), ()))
    ll_r = jax.lax.dot_general(red, dl * dl, dn,
                               preferred_element_type=jnp.float32)
    rl_r = jax.lax.dot_general(red, dr * dr, dn,
                               preferred_element_type=jnp.float32)

    a_ab[...] += v_r * wi
    a_lc[...] += v_r * wl
    a_rc[...] += v_r * wr
    a_ll[...] += ll_r * wl
    a_rl[...] += rl_r * wr
    a_wi[...] += wi
    a_wl[...] += wl
    a_wr[...] += wr

    @pl.when((g == _NG - 1) & (j == _NJ - 1))
    def _final():
        s_ab = jnp.sum(a_ab[...])
        s_lc = jnp.sum(a_lc[...])
        s_rc = jnp.sum(a_rc[...])
        s_ll = jnp.sum(a_ll[...])
        s_rl = jnp.sum(a_rl[...])
        n_i = jnp.sum(a_wi[...])
        n_l = jnp.sum(a_wl[...])
        n_r = jnp.sum(a_wr[...])

        def mmean(s, n):
            return jnp.where(n > 0, s / jnp.maximum(n, 1.0), jnp.float32(0.0))

        out_ref[0, 0] = (mmean(s_ab, n_i) * 0.5
                         + mmean(s_lc, n_l) + mmean(s_ll, n_l) * 0.25
                         + mmean(s_rc, n_r) + mmean(s_rl, n_r) * 0.25)


@functools.partial(jax.jit, static_argnames=())
def kernel(conf, conf_flip, loc, loc_flip, conf_shuffle, conf_interpolation,
           loc_shuffle, loc_interpolation, lam):
    del conf_flip, loc_flip
    lam_s = jnp.reshape(lam.astype(jnp.float32), (1, 1))

    cT = jnp.transpose(conf, (2, 0, 1))
    tT = jnp.transpose(conf_shuffle, (2, 0, 1))
    iT = jnp.transpose(conf_interpolation, (2, 0, 1))
    loT = jnp.transpose(loc, (0, 2, 1))
    lsT = jnp.transpose(loc_shuffle, (0, 2, 1))
    liT = jnp.transpose(loc_interpolation, (0, 2, 1))

    half_g = (_B // 2) // _GB
    conf_spec = pl.BlockSpec((_C, _GB, _PBLK), lambda g, j: (0, g, j))
    swap_spec = pl.BlockSpec((_C, _GB, _PBLK),
                             lambda g, j: (0, (g + half_g) % _NG, j))
    loc_spec = pl.BlockSpec((_GB, 4, _PBLK), lambda g, j: (g, 0, j))
    lswap_spec = pl.BlockSpec((_GB, 4, _PBLK),
                              lambda g, j: ((g + half_g) % _NG, 0, j))

    out = pl.pallas_call(
        _body,
        grid=(_NG, _NJ),
        in_specs=[
            pl.BlockSpec(memory_space=pltpu.SMEM),
            conf_spec, swap_spec, conf_spec,
            loc_spec, lswap_spec, loc_spec,
        ],
        out_specs=pl.BlockSpec(memory_space=pltpu.SMEM),
        out_shape=jax.ShapeDtypeStruct((1, 1), jnp.float32),
        scratch_shapes=[pltpu.VMEM((_GB, _PBLK), jnp.float32)] * 8,
        compiler_params=pltpu.CompilerParams(
            dimension_semantics=("arbitrary", "arbitrary"),
        ),
    )(lam_s, cT, tT, iT, loT, lsT, liT)

    return out[0, 0]


# speedup vs baseline: 1.1306x; 1.1306x over previous
import functools

import jax
import jax.numpy as jnp
from jax.experimental import pallas as pl
from jax.experimental.pallas import tpu as pltpu

_B, _P, _C = 32, 8732, 21
_PBLK = 2944
_NJ = (_P + _PBLK - 1) // _PBLK
_GB = 8
_NG = _B // _GB
_EPS = 1e-7


def _body(lam_ref, c_ref, t_ref, ci_ref, lo_ref, ls_ref, li_ref,
          out_ref,
          a_ab, a_lc, a_rc, a_ll, a_rl, a_wi, a_wl, a_wr):
    g = pl.program_id(0)
    j = pl.program_id(1)

    @pl.when((g == 0) & (j == 0))
    def _init():
        for a in (a_ab, a_lc, a_rc, a_ll, a_rl, a_wi, a_wl, a_wr):
            a[...] = jnp.zeros_like(a)

    lam = lam_ref[0, 0]
    lane = jax.lax.broadcasted_iota(jnp.int32, (1, _PBLK), 1)
    valid = (j * _PBLK + lane) < _P


    c0 = c_ref[0]
    t0 = t_ref[0]
    cmax = c0
    tmax = t0
    for k in range(1, _C):
        cmax = jnp.maximum(cmax, c_ref[k])
        tmax = jnp.maximum(tmax, t_ref[k])
    left = cmax > c0
    right = tmax > t0
    bi = left & right & valid
    bl = left & ~right & valid
    br = right & ~left & valid
    wi = bi.astype(jnp.float32)
    wl = bl.astype(jnp.float32)
    wr = br.astype(jnp.float32)

    bx = bl | br
    v_r = jnp.zeros((_GB, _PBLK), jnp.float32)
    for k in range(_C):
        ck = c_ref[k]
        tk = t_ref[k]
        ins = jnp.where(valid, ci_ref[k], 0.5) + _EPS
        mixed = lam * ck + (1.0 - lam) * tk + _EPS
        den = jnp.where(bi, mixed, ins)
        num = jnp.where(bl, ck + _EPS, jnp.where(br, tk + _EPS, ins))
        coef = jnp.where(bi, ins - mixed, jnp.where(bx, num, 0.0))
        v_r = v_r + coef * jnp.log(num / den)

    lo = lo_ref[...].reshape(4 * _GB, _PBLK)
    ls = ls_ref[...].reshape(4 * _GB, _PBLK)
    li = li_ref[...].reshape(4 * _GB, _PBLK)
    dl = jnp.where(valid, li - lo, 0.0)
    dr = jnp.where(valid, li - ls, 0.0)
    ri = jax.lax.broadcasted_iota(jnp.int32, (_GB, 4 * _GB), 0)
    cj = jax.lax.broadcasted_iota(jnp.int32, (_GB, 4 * _GB), 1)
    red = ((cj // 4) == ri).astype(jnp.float32)
    dn = (((1,), (0,)), ((), ()))
    ll_r = jax.lax.dot_general(red, dl * dl, dn,
                               preferred_element_type=jnp.float32)
    rl_r = jax.lax.dot_general(red, dr * dr, dn,
                               preferred_element_type=jnp.float32)

    a_ab[...] += v_r * wi
    a_lc[...] += v_r * wl
    a_rc[...] += v_r * wr
    a_ll[...] += ll_r * wl
    a_rl[...] += rl_r * wr
    a_wi[...] += wi
    a_wl[...] += wl
    a_wr[...] += wr

    @pl.when((g == _NG - 1) & (j == _NJ - 1))
    def _final():
        s_ab = jnp.sum(a_ab[...])
        s_lc = jnp.sum(a_lc[...])
        s_rc = jnp.sum(a_rc[...])
        s_ll = jnp.sum(a_ll[...])
        s_rl = jnp.sum(a_rl[...])
        n_i = jnp.sum(a_wi[...])
        n_l = jnp.sum(a_wl[...])
        n_r = jnp.sum(a_wr[...])

        def mmean(s, n):
            return jnp.where(n > 0, s / jnp.maximum(n, 1.0), jnp.float32(0.0))

        out_ref[0, 0] = (mmean(s_ab, n_i) * 0.5
                         + mmean(s_lc, n_l) + mmean(s_ll, n_l) * 0.25
                         + mmean(s_rc, n_r) + mmean(s_rl, n_r) * 0.25)


@functools.partial(jax.jit, static_argnames=())
def kernel(conf, conf_flip, loc, loc_flip, conf_shuffle, conf_interpolation,
           loc_shuffle, loc_interpolation, lam):
    del conf_flip, loc_flip
    lam_s = jnp.reshape(lam.astype(jnp.float32), (1, 1))

    cT = jnp.transpose(conf, (2, 0, 1))
    tT = jnp.transpose(conf_shuffle, (2, 0, 1))
    iT = jnp.transpose(conf_interpolation, (2, 0, 1))
    loT = jnp.transpose(loc, (0, 2, 1))
    lsT = jnp.transpose(loc_shuffle, (0, 2, 1))
    liT = jnp.transpose(loc_interpolation, (0, 2, 1))

    half_g = (_B // 2) // _GB
    conf_spec = pl.BlockSpec((_C, _GB, _PBLK), lambda g, j: (0, g, j))
    swap_spec = pl.BlockSpec((_C, _GB, _PBLK),
                             lambda g, j: (0, (g + half_g) % _NG, j))
    loc_spec = pl.BlockSpec((_GB, 4, _PBLK), lambda g, j: (g, 0, j))
    lswap_spec = pl.BlockSpec((_GB, 4, _PBLK),
                              lambda g, j: ((g + half_g) % _NG, 0, j))

    out = pl.pallas_call(
        _body,
        grid=(_NG, _NJ),
        in_specs=[
            pl.BlockSpec(memory_space=pltpu.SMEM),
            conf_spec, swap_spec, conf_spec,
            loc_spec, lswap_spec, loc_spec,
        ],
        out_specs=pl.BlockSpec(memory_space=pltpu.SMEM),
        out_shape=jax.ShapeDtypeStruct((1, 1), jnp.float32),
        scratch_shapes=[pltpu.VMEM((_GB, _PBLK), jnp.float32)] * 8,
        compiler_params=pltpu.CompilerParams(
            dimension_semantics=("arbitrary", "arbitrary"),
        ),
    )(lam_s, cT, tT, iT, loT, lsT, liT)

    return out[0, 0]
